# baseline (speedup 1.0000x reference)
import functools

import jax
import jax.numpy as jnp
from jax import lax
from jax.experimental import pallas as pl
from jax.experimental.pallas import tpu as pltpu
from jax.experimental.pallas import tpu_sc as plsc

B, C, H, W = 8, 96, 64, 64
HW = H * W
ND = 4
CP = C + 16
K = 128
NCH = HW // K
SLAB = HW + K
QR = HW // 4
MR = K // ND
PT = 2048
NJ = C // 16
NB = 4


def _mm_body(x_ref, xt_ref, w_ref, o_ref):
    wt = w_ref[0]
    dd = pl.program_id(2)

    @pl.when(dd < 2)
    def _():
        o_ref[0, 0] = lax.dot_general(
            x_ref[0], wt, (((0,), (0,)), ((), ())),
            preferred_element_type=jnp.float32)

    @pl.when(dd >= 2)
    def _():
        o_ref[0, 0] = lax.dot_general(
            xt_ref[0], wt, (((0,), (0,)), ((), ())),
            preferred_element_type=jnp.float32)


def _project(x_flat, xt_flat, wdt):
    return pl.pallas_call(
        _mm_body,
        grid=(B, HW // PT, ND),
        in_specs=[
            pl.BlockSpec((1, C, PT), lambda b, t, d: (b, 0, t)),
            pl.BlockSpec((1, C, PT), lambda b, t, d: (b, 0, t)),
            pl.BlockSpec((1, C, C), lambda b, t, d: (d, 0, 0)),
        ],
        out_specs=pl.BlockSpec((1, 1, PT, C), lambda b, t, d: (b, d, t, 0)),
        out_shape=jax.ShapeDtypeStruct((B, ND, HW, C), jnp.float32),
        compiler_params=pltpu.CompilerParams(fuse_transposed_lhs_in_matmul=True),
    )(x_flat, xt_flat, wdt)


def _sc_body(xu_hbm, idxg_hbm, idxr_hbm, la_hbm,
             acc_hbm, out_hbm, inv_hbm,
             idxg_v, idxr_v, u_buf, ys_buf, la_v, a_v, inv_ref, inv4_v,
             sem_g, sem_s, sem_z, sem_o):
    c = lax.axis_index("c")
    s = lax.axis_index("s")
    b = c * 4 + s // 4
    d = s % 4
    q = s % 4
    w = b * ND + d
    wid = c * 16 + s
    sbase = w * SLAB

    with jax.named_scope("sc_setup_zero"):
        pltpu.sync_copy(idxg_hbm.at[w], idxg_v)
        pltpu.sync_copy(idxr_hbm.at[w], idxr_v)
        pltpu.sync_copy(la_hbm.at[d], la_v)

        hwvec = jnp.full((16,), HW, jnp.int32)

        def irow(i, _):
            inv_ref[pl.ds(i * 16, 16)] = hwvec
            return 0
        lax.fori_loop(0, (HW + 16) // 16, irow, 0)

        base16 = jnp.arange(16, dtype=jnp.int32)

        def rrow(t, _):
            row = t // (K // 16)
            jj = t % (K // 16)
            v = idxr_v[row, pl.ds(jj * 16, 16)]
            plsc.store_scatter(inv_ref, [v], base16 + t * 16)
            return 0
        lax.fori_loop(0, NCH * (K // 16), rrow, 0)
        pltpu.sync_copy(inv_ref.at[pl.ds(0, HW)], inv_hbm.at[w])

        zvec = jnp.zeros((16,), jnp.float32)

        def zrow(i, _):
            for j in range(CP // 16):
                ys_buf[0, i, pl.ds(j * 16, 16)] = zvec
            return 0
        lax.fori_loop(0, K, zrow, 0)
        pltpu.async_copy(ys_buf.at[0], acc_hbm.at[pl.ds(sbase + HW, K)], sem_z)

        for j in range(NJ):
            v = la_v[pl.ds(j * 16, 16)]
            a_v[pl.ds(j * 16, 16)] = 1.0 / (1.0 + jnp.exp(-v))

        ovec = jnp.ones((16,), jnp.float32)

        def prow(i, _):
            for n in range(1, NB):
                ys_buf[n, i, pl.ds(C, 16)] = ovec
            return 0
        lax.fori_loop(0, K, prow, 0)

        pltpu.make_async_copy(
            ys_buf.at[0], acc_hbm.at[pl.ds(sbase + HW, K)], sem_z).wait()

        def prow0(i, _):
            ys_buf[0, i, pl.ds(C, 16)] = ovec
            return 0
        lax.fori_loop(0, K, prow0, 0)

    a_regs = [a_v[pl.ds(j * 16, 16)] for j in range(NJ)]

    for n in range(NB - 1):
        pltpu.async_copy(xu_hbm.at[idxg_v.at[n]], u_buf.at[n], sem_g)

    NGO = NCH // NB

    def chunk4(o, h):
        for i in range(NB):
            g = o * NB + i
            pltpu.make_async_copy(
                xu_hbm.at[idxg_v.at[g]], u_buf.at[i], sem_g).wait()
            if i == 0:
                pltpu.async_copy(
                    xu_hbm.at[idxg_v.at[g + NB - 1]],
                    u_buf.at[NB - 1], sem_g)
            else:
                @pl.when(g + NB - 1 < NCH)
                def _():
                    pltpu.async_copy(
                        xu_hbm.at[idxg_v.at[g + NB - 1]],
                        u_buf.at[i - 1], sem_g)

            @pl.when(o >= 1)
            def _():
                pltpu.make_async_copy(
                    ys_buf.at[i],
                    acc_hbm.at[pl.ds(sbase + (g - NB) * K, K)],
                    sem_s).wait()

            def srow(l, hh):
                hs = []
                for j in range(NJ):
                    u = u_buf[i, l, pl.ds(j * 16, 16)]
                    nh = a_regs[j] * hh[j] + u
                    ys_buf[i, l, pl.ds(j * 16, 16)] = nh
                    hs.append(nh)
                return tuple(hs)
            h = lax.fori_loop(0, K, srow, h)

            pltpu.async_copy(
                ys_buf.at[i], acc_hbm.at[pl.ds(sbase + g * K, K)], sem_s)
        return h

    h0 = tuple(jnp.zeros((16,), jnp.float32) for _ in range(NJ))
    with jax.named_scope("sc_mainloop"):
        lax.fori_loop(0, NGO, chunk4, h0)

        for i in range(NB):
            g = NCH - NB + i
            pltpu.make_async_copy(
                ys_buf.at[i], acc_hbm.at[pl.ds(sbase + g * K, K)],
                sem_s).wait()

    with jax.named_scope("sc_barrier"):
        plsc.subcore_barrier()

    jax.named_scope("sc_merge").__enter__()
    for dd in range(ND):
        pltpu.sync_copy(inv_hbm.at[b * ND + dd, pl.ds(q * QR, QR)],
                        inv4_v.at[dd])

    def mtab(m, _):
        for dd in range(ND):
            off = (b * ND + dd) * SLAB
            for v2 in range(MR // 16):
                val = inv4_v[dd, pl.ds(m * MR + v2 * 16, 16)] + off
                idxg_v[m, pl.ds(dd * MR + v2 * 16, 16)] = val
        return 0
    lax.fori_loop(0, QR // MR, mtab, 0)
    obase = q * QR
    NM = QR // MR
    NMO = NM // NB

    for n in range(NB - 1):
        pltpu.async_copy(acc_hbm.at[idxg_v.at[n]], ys_buf.at[n], sem_g)

    def merge4(o, _):
        for i in range(NB):
            m = o * NB + i
            pltpu.make_async_copy(
                acc_hbm.at[idxg_v.at[m]], ys_buf.at[i], sem_g).wait()
            if i == 0:
                pltpu.async_copy(
                    acc_hbm.at[idxg_v.at[m + NB - 1]],
                    ys_buf.at[NB - 1], sem_g)
            else:
                @pl.when(m + NB - 1 < NM)
                def _():
                    pltpu.async_copy(
                        acc_hbm.at[idxg_v.at[m + NB - 1]],
                        ys_buf.at[i - 1], sem_g)

            @pl.when(o >= 1)
            def _():
                pltpu.make_async_copy(
                    u_buf.at[i].at[pl.ds(0, MR)],
                    out_hbm.at[b, pl.ds(obase + (m - NB) * MR, MR)],
                    sem_o).wait()

            def mrow(r, _2):
                cnt = (ys_buf[i, r, pl.ds(C, 16)]
                       + ys_buf[i, MR + r, pl.ds(C, 16)]
                       + ys_buf[i, 2 * MR + r, pl.ds(C, 16)]
                       + ys_buf[i, 3 * MR + r, pl.ds(C, 16)])
                inv = 1.0 / (cnt + 1e-6)
                for j in range(NJ):
                    tot = (ys_buf[i, r, pl.ds(j * 16, 16)]
                           + ys_buf[i, MR + r, pl.ds(j * 16, 16)]
                           + ys_buf[i, 2 * MR + r, pl.ds(j * 16, 16)]
                           + ys_buf[i, 3 * MR + r, pl.ds(j * 16, 16)])
                    u_buf[i, r, pl.ds(j * 16, 16)] = tot * inv
                return 0
            lax.fori_loop(0, MR, mrow, 0)

            pltpu.async_copy(
                u_buf.at[i].at[pl.ds(0, MR)],
                out_hbm.at[b, pl.ds(obase + m * MR, MR)],
                sem_o)
        return 0

    lax.fori_loop(0, NMO, merge4, 0)

    for i in range(NB):
        pltpu.make_async_copy(
            u_buf.at[i].at[pl.ds(0, MR)],
            out_hbm.at[b, pl.ds(obase + (NM - NB + i) * MR, MR)],
            sem_o).wait()


@functools.partial(
    pl.kernel,
    out_type=(
        jax.ShapeDtypeStruct((B * ND * SLAB, CP), jnp.float32),
        jax.ShapeDtypeStruct((B, HW, C), jnp.float32),
        jax.ShapeDtypeStruct((B * ND, HW), jnp.int32),
    ),
    mesh=plsc.VectorSubcoreMesh(core_axis_name="c", subcore_axis_name="s"),
    scratch_types=[
        pltpu.VMEM((NCH, K), jnp.int32),
        pltpu.VMEM((NCH, K), jnp.int32),
        pltpu.VMEM((NB, K, C), jnp.float32),
        pltpu.VMEM((NB, K, CP), jnp.float32),
        pltpu.VMEM((C,), jnp.float32),
        pltpu.VMEM((C,), jnp.float32),
        pltpu.VMEM((HW + 16,), jnp.int32),
        pltpu.VMEM((ND, QR), jnp.int32),
        pltpu.SemaphoreType.DMA,
        pltpu.SemaphoreType.DMA,
        pltpu.SemaphoreType.DMA,
        pltpu.SemaphoreType.DMA,
    ],
    compiler_params=pltpu.CompilerParams(
        use_tc_tiling_on_sc=False, needs_layout_passes=False),
)
def _sc_ragged(xu_flat, idxg, idxr, log_a, acc, out, inv, *scratch):
    _sc_body(xu_flat, idxg, idxr, log_a, acc, out, inv, *scratch)


def kernel(x, Wd, log_a, scan_idx, mask):
    x_flat = x.reshape(B, C, HW)
    xt_flat = jnp.swapaxes(x, 2, 3).reshape(B, C, HW)
    wdt = jnp.transpose(Wd, (0, 2, 1))
    xu = _project(x_flat, xt_flat, wdt)
    xu_flat = xu.reshape(B * ND * HW, C)

    woff = (jnp.arange(B, dtype=jnp.int32) * ND)[:, None] \
        + jnp.arange(ND, dtype=jnp.int32)[None, :]
    clamped = jnp.minimum(scan_idx, HW - 1)
    transposed = (clamped % W) * H + clamped // W
    pix = jnp.where(jnp.arange(ND)[None, :, None] >= 2, transposed, clamped)
    idxg = pix + (woff * HW)[:, :, None]
    del mask

    _, out, _ = _sc_ragged(
        xu_flat,
        idxg.reshape(B * ND, NCH, K),
        scan_idx.reshape(B * ND, NCH, K),
        log_a,
    )
    return jnp.transpose(out.reshape(B, H, W, C), (0, 3, 1, 2))


# speedup vs baseline: 1.0627x; 1.0627x over previous
import functools

import jax
import jax.numpy as jnp
from jax import lax
from jax.experimental import pallas as pl
from jax.experimental.pallas import tpu as pltpu
from jax.experimental.pallas import tpu_sc as plsc

B, C, H, W = 8, 96, 64, 64
HW = H * W
ND = 4
CP = C + 16
K = 128
NCH = HW // K
SLAB = HW + K
QR = HW // 4
MR = K // ND
PT = 2048
NJ = C // 16
NB = 4


def _mm_body(x_ref, w_ref, o_ref):
    xb = x_ref[0]
    wt = w_ref[0]
    o_ref[0, 0] = lax.dot_general(
        xb, wt, (((0,), (0,)), ((), ())), preferred_element_type=jnp.float32)


def _project(x_flat, wdt):
    return pl.pallas_call(
        _mm_body,
        grid=(B, HW // PT, ND),
        in_specs=[
            pl.BlockSpec((1, C, PT), lambda b, t, d: (b, 0, t)),
            pl.BlockSpec((1, C, C), lambda b, t, d: (d, 0, 0)),
        ],
        out_specs=pl.BlockSpec((1, 1, PT, C), lambda b, t, d: (b, d, t, 0)),
        out_shape=jax.ShapeDtypeStruct((B, ND, HW, C), jnp.float32),
        compiler_params=pltpu.CompilerParams(fuse_transposed_lhs_in_matmul=True),
    )(x_flat, wdt)


def _sc_body(xu_hbm, idxg_hbm, idxr_hbm, la_hbm,
             acc_hbm, out_hbm, inv_hbm,
             idxg_v, idxr_v, u_buf, ys_buf, la_v, a_v, inv_ref, inv4_v,
             sem_g, sem_s, sem_z, sem_o):
    c = lax.axis_index("c")
    s = lax.axis_index("s")
    b = c * 4 + s // 4
    d = s % 4
    q = s % 4
    w = b * ND + d
    wid = c * 16 + s
    sbase = w * SLAB

    with jax.named_scope("sc_setup_zero"):
        pltpu.sync_copy(idxg_hbm.at[w], idxg_v)
        pltpu.sync_copy(idxr_hbm.at[w], idxr_v)
        pltpu.sync_copy(la_hbm.at[d], la_v)

        hwvec = jnp.full((16,), HW, jnp.int32)

        def irow(i, _):
            inv_ref[pl.ds(i * 16, 16)] = hwvec
            return 0
        lax.fori_loop(0, (HW + 16) // 16, irow, 0)

        base16 = jnp.arange(16, dtype=jnp.int32)

        def rrow(t, _):
            row = t // (K // 16)
            jj = t % (K // 16)
            v = idxr_v[row, pl.ds(jj * 16, 16)]
            plsc.store_scatter(inv_ref, [v], base16 + t * 16)
            return 0
        lax.fori_loop(0, NCH * (K // 16), rrow, 0)
        pltpu.sync_copy(inv_ref.at[pl.ds(0, HW)], inv_hbm.at[w])

        zvec = jnp.zeros((16,), jnp.float32)

        def zrow(i, _):
            for j in range(CP // 16):
                ys_buf[0, i, pl.ds(j * 16, 16)] = zvec
            return 0
        lax.fori_loop(0, K, zrow, 0)
        pltpu.async_copy(ys_buf.at[0], acc_hbm.at[pl.ds(sbase + HW, K)], sem_z)

        for j in range(NJ):
            v = la_v[pl.ds(j * 16, 16)]
            a_v[pl.ds(j * 16, 16)] = 1.0 / (1.0 + jnp.exp(-v))

        ovec = jnp.ones((16,), jnp.float32)

        def prow(i, _):
            for n in range(1, NB):
                ys_buf[n, i, pl.ds(C, 16)] = ovec
            return 0
        lax.fori_loop(0, K, prow, 0)

        pltpu.make_async_copy(
            ys_buf.at[0], acc_hbm.at[pl.ds(sbase + HW, K)], sem_z).wait()

        def prow0(i, _):
            ys_buf[0, i, pl.ds(C, 16)] = ovec
            return 0
        lax.fori_loop(0, K, prow0, 0)

    a_regs = [a_v[pl.ds(j * 16, 16)] for j in range(NJ)]

    for n in range(NB - 1):
        pltpu.async_copy(xu_hbm.at[idxg_v.at[n]], u_buf.at[n], sem_g)

    NGO = NCH // NB

    def chunk4(o, h):
        for i in range(NB):
            g = o * NB + i
            pltpu.make_async_copy(
                xu_hbm.at[idxg_v.at[g]], u_buf.at[i], sem_g).wait()
            if i == 0:
                pltpu.async_copy(
                    xu_hbm.at[idxg_v.at[g + NB - 1]],
                    u_buf.at[NB - 1], sem_g)
            else:
                @pl.when(g + NB - 1 < NCH)
                def _():
                    pltpu.async_copy(
                        xu_hbm.at[idxg_v.at[g + NB - 1]],
                        u_buf.at[i - 1], sem_g)

            @pl.when(o >= 1)
            def _():
                pltpu.make_async_copy(
                    ys_buf.at[i],
                    acc_hbm.at[pl.ds(sbase + (g - NB) * K, K)],
                    sem_s).wait()

            def srow(l, hh):
                hs = []
                for j in range(NJ):
                    u = u_buf[i, l, pl.ds(j * 16, 16)]
                    nh = a_regs[j] * hh[j] + u
                    ys_buf[i, l, pl.ds(j * 16, 16)] = nh
                    hs.append(nh)
                return tuple(hs)
            h = lax.fori_loop(0, K, srow, h)

            pltpu.async_copy(
                ys_buf.at[i], acc_hbm.at[pl.ds(sbase + g * K, K)], sem_s)
        return h

    h0 = tuple(jnp.zeros((16,), jnp.float32) for _ in range(NJ))
    with jax.named_scope("sc_mainloop"):
        lax.fori_loop(0, NGO, chunk4, h0)

        for i in range(NB):
            g = NCH - NB + i
            pltpu.make_async_copy(
                ys_buf.at[i], acc_hbm.at[pl.ds(sbase + g * K, K)],
                sem_s).wait()

    with jax.named_scope("sc_barrier"):
        plsc.subcore_barrier()

    jax.named_scope("sc_merge").__enter__()
    for dd in range(ND):
        pltpu.sync_copy(inv_hbm.at[b * ND + dd, pl.ds(q * QR, QR)],
                        inv4_v.at[dd])

    def mtab(m, _):
        for dd in range(ND):
            off = (b * ND + dd) * SLAB
            for v2 in range(MR // 16):
                val = inv4_v[dd, pl.ds(m * MR + v2 * 16, 16)] + off
                idxg_v[m, pl.ds(dd * MR + v2 * 16, 16)] = val
        return 0
    lax.fori_loop(0, QR // MR, mtab, 0)
    obase = q * QR
    NM = QR // MR
    NMO = NM // NB

    for n in range(NB - 1):
        pltpu.async_copy(acc_hbm.at[idxg_v.at[n]], ys_buf.at[n], sem_g)

    def merge4(o, _):
        for i in range(NB):
            m = o * NB + i
            pltpu.make_async_copy(
                acc_hbm.at[idxg_v.at[m]], ys_buf.at[i], sem_g).wait()
            if i == 0:
                pltpu.async_copy(
                    acc_hbm.at[idxg_v.at[m + NB - 1]],
                    ys_buf.at[NB - 1], sem_g)
            else:
                @pl.when(m + NB - 1 < NM)
                def _():
                    pltpu.async_copy(
                        acc_hbm.at[idxg_v.at[m + NB - 1]],
                        ys_buf.at[i - 1], sem_g)

            @pl.when(o >= 1)
            def _():
                pltpu.make_async_copy(
                    u_buf.at[i].at[pl.ds(0, MR)],
                    out_hbm.at[b, pl.ds(obase + (m - NB) * MR, MR)],
                    sem_o).wait()

            def mrow(r, _2):
                cnt = (ys_buf[i, r, pl.ds(C, 16)]
                       + ys_buf[i, MR + r, pl.ds(C, 16)]
                       + ys_buf[i, 2 * MR + r, pl.ds(C, 16)]
                       + ys_buf[i, 3 * MR + r, pl.ds(C, 16)])
                inv = 1.0 / (cnt + 1e-6)
                for j in range(NJ):
                    tot = (ys_buf[i, r, pl.ds(j * 16, 16)]
                           + ys_buf[i, MR + r, pl.ds(j * 16, 16)]
                           + ys_buf[i, 2 * MR + r, pl.ds(j * 16, 16)]
                           + ys_buf[i, 3 * MR + r, pl.ds(j * 16, 16)])
                    u_buf[i, r, pl.ds(j * 16, 16)] = tot * inv
                return 0
            lax.fori_loop(0, MR, mrow, 0)

            pltpu.async_copy(
                u_buf.at[i].at[pl.ds(0, MR)],
                out_hbm.at[b, pl.ds(obase + m * MR, MR)],
                sem_o)
        return 0

    lax.fori_loop(0, NMO, merge4, 0)

    for i in range(NB):
        pltpu.make_async_copy(
            u_buf.at[i].at[pl.ds(0, MR)],
            out_hbm.at[b, pl.ds(obase + (NM - NB + i) * MR, MR)],
            sem_o).wait()


@functools.partial(
    pl.kernel,
    out_type=(
        jax.ShapeDtypeStruct((B * ND * SLAB, CP), jnp.float32),
        jax.ShapeDtypeStruct((B, HW, C), jnp.float32),
        jax.ShapeDtypeStruct((B * ND, HW), jnp.int32),
    ),
    mesh=plsc.VectorSubcoreMesh(core_axis_name="c", subcore_axis_name="s"),
    scratch_types=[
        pltpu.VMEM((NCH, K), jnp.int32),
        pltpu.VMEM((NCH, K), jnp.int32),
        pltpu.VMEM((NB, K, C), jnp.float32),
        pltpu.VMEM((NB, K, CP), jnp.float32),
        pltpu.VMEM((C,), jnp.float32),
        pltpu.VMEM((C,), jnp.float32),
        pltpu.VMEM((HW + 16,), jnp.int32),
        pltpu.VMEM((ND, QR), jnp.int32),
        pltpu.SemaphoreType.DMA,
        pltpu.SemaphoreType.DMA,
        pltpu.SemaphoreType.DMA,
        pltpu.SemaphoreType.DMA,
    ],
    compiler_params=pltpu.CompilerParams(
        use_tc_tiling_on_sc=False, needs_layout_passes=False),
)
def _sc_ragged(xu_flat, idxg, idxr, log_a, acc, out, inv, *scratch):
    _sc_body(xu_flat, idxg, idxr, log_a, acc, out, inv, *scratch)


def kernel(x, Wd, log_a, scan_idx, mask):
    x_flat = x.reshape(B, C, HW)
    wdt = jnp.transpose(Wd, (0, 2, 1))
    xu = _project(x_flat, wdt)
    xu_flat = xu.reshape(B * ND * HW, C)

    woff = (jnp.arange(B, dtype=jnp.int32) * ND)[:, None] \
        + jnp.arange(ND, dtype=jnp.int32)[None, :]
    idxg = jnp.minimum(scan_idx, HW - 1) + (woff * HW)[:, :, None]
    del mask

    _, out, _ = _sc_ragged(
        xu_flat,
        idxg.reshape(B * ND, NCH, K),
        scan_idx.reshape(B * ND, NCH, K),
        log_a,
    )
    return jnp.transpose(out.reshape(B, H, W, C), (0, 3, 1, 2))
